# baseline (speedup 1.0000x reference)
import functools

import jax
import jax.numpy as jnp
from jax import lax
from jax.experimental import pallas as pl
from jax.experimental.pallas import tpu as pltpu
from jax.experimental.pallas import tpu_sc as plsc

N = 10000
E = 160000
D = 256
H = 8
C = 32
K = 200

NPAD = 10240
NC, NS, L = 2, 16, 16
NW = NC * NS

_SC_MESH = plsc.VectorSubcoreMesh(core_axis_name="c", subcore_axis_name="s",
                                  num_cores=NC)
_SC_PARAMS = pltpu.CompilerParams(needs_layout_passes=False)


def _zero_vmem(ref, nwords):
    z = jnp.zeros((L,), ref.dtype)

    def body(i, _):
        ref[pl.ds(i * L, L)] = z
        return 0

    lax.fori_loop(0, nwords // L, body, 0)


_EPW = E // NW
_EPW_PAD = _EPW + 128 - (_EPW % 128)
_ACC_PAD = NPAD + 128


@functools.partial(
    pl.kernel,
    out_type=jax.ShapeDtypeStruct((NW, NPAD), jnp.float32),
    mesh=_SC_MESH,
    scratch_types=[
        pltpu.VMEM((_EPW_PAD,), jnp.int32),
        pltpu.VMEM((_ACC_PAD,), jnp.float32),
    ],
    compiler_params=_SC_PARAMS,
)
def _sc_deg(dst_hbm, out_hbm, idx_v, acc_v):
    wid = lax.axis_index("s") * NC + lax.axis_index("c")
    base = wid * _EPW
    dummy = jnp.full((L,), NPAD, jnp.int32)
    _tail0 = (_EPW // L) * L

    def fill(i, _):
        idx_v[pl.ds(_tail0 + i * L, L)] = dummy
        return 0

    lax.fori_loop(0, (_EPW_PAD - _tail0) // L, fill, 0)
    pltpu.sync_copy(dst_hbm.at[pl.ds(base, _EPW)], idx_v.at[pl.ds(0, _EPW)])
    _zero_vmem(acc_v, _ACC_PAD)
    ones = jnp.full((L,), 1.0, jnp.float32)

    def body(i, _):
        for u in range(4):
            idx = idx_v[pl.ds((i * 4 + u) * L, L)]
            plsc.addupdate_scatter(acc_v, [idx], ones)
        return 0

    lax.fori_loop(0, _EPW_PAD // L // 4, body, 0)
    pltpu.sync_copy(acc_v.at[pl.ds(0, NPAD)], out_hbm.at[wid])


_ECH = E // (NW // 2)


@functools.partial(
    pl.kernel,
    out_type=jax.ShapeDtypeStruct((NW, 4, NPAD), jnp.float32),
    mesh=_SC_MESH,
    scratch_types=[
        pltpu.VMEM((10240,), jnp.int32),
        pltpu.VMEM((10240,), jnp.int32),
        pltpu.VMEM((4, NPAD), jnp.float32),
        pltpu.VMEM((4, NPAD + 128), jnp.float32),
    ],
    compiler_params=_SC_PARAMS,
)
def _sc_msg(src_hbm, dst_hbm, g_hbm, out_hbm, src_v, dst_v, g4_v, acc_v):
    wid = lax.axis_index("s") * NC + lax.axis_index("c")
    g2 = wid % 2
    chunk = wid // 2
    base = chunk * _ECH
    zvec = jnp.zeros((L,), jnp.int32)
    dummy = jnp.full((L,), NPAD, jnp.int32)

    def fillt(i, _):
        src_v[pl.ds(_ECH + i * L, L)] = zvec
        dst_v[pl.ds(_ECH + i * L, L)] = dummy
        return 0

    lax.fori_loop(0, (10240 - _ECH) // L, fillt, 0)
    pltpu.sync_copy(src_hbm.at[pl.ds(base, _ECH)], src_v.at[pl.ds(0, _ECH)])
    pltpu.sync_copy(dst_hbm.at[pl.ds(base, _ECH)], dst_v.at[pl.ds(0, _ECH)])
    pltpu.sync_copy(g_hbm.at[pl.ds(4 * g2, 4), :], g4_v)
    z = jnp.zeros((L,), jnp.float32)

    def zbody(i, _):
        acc_v[i // ((NPAD + 128) // L), pl.ds((i % ((NPAD + 128) // L)) * L, L)] = z
        return 0

    lax.fori_loop(0, 4 * ((NPAD + 128) // L), zbody, 0)

    fvecs = [jnp.full((L,), f, jnp.int32) for f in range(4)]

    def body(i, _):
        for u in range(4):
            sv = src_v[pl.ds((i * 4 + u) * L, L)]
            dv = dst_v[pl.ds((i * 4 + u) * L, L)]
            for f in range(4):
                val = plsc.load_gather(g4_v, [fvecs[f], sv])
                plsc.addupdate_scatter(acc_v, [fvecs[f], dv], val)
        return 0

    lax.fori_loop(0, 10240 // L // 4, body, 0)
    pltpu.sync_copy(acc_v.at[:, pl.ds(0, NPAD)], out_hbm.at[wid])


_A2 = 208


@functools.partial(
    pl.kernel,
    out_type=jax.ShapeDtypeStruct((NW, _A2, _A2), jnp.float32),
    mesh=_SC_MESH,
    scratch_types=[
        pltpu.VMEM((_EPW_PAD,), jnp.int32),
        pltpu.VMEM((_EPW_PAD,), jnp.int32),
        pltpu.VMEM((NPAD + 128,), jnp.int32),
        pltpu.VMEM((256,), jnp.int32),
        pltpu.VMEM((_A2, _A2), jnp.float32),
    ],
    compiler_params=_SC_PARAMS,
)
def _sc_conv2(src_hbm, dst_hbm, perm_hbm, out_hbm, src_v, dst_v, nn_v, perm_v,
              a2_v):
    wid = lax.axis_index("s") * NC + lax.axis_index("c")
    base = wid * _EPW
    dummy = jnp.full((L,), NPAD, jnp.int32)
    _tail0 = (_EPW // L) * L

    def fill(i, _):
        src_v[pl.ds(_tail0 + i * L, L)] = dummy
        dst_v[pl.ds(_tail0 + i * L, L)] = dummy
        return 0

    lax.fori_loop(0, (_EPW_PAD - _tail0) // L, fill, 0)
    pltpu.sync_copy(src_hbm.at[pl.ds(base, _EPW)], src_v.at[pl.ds(0, _EPW)])
    pltpu.sync_copy(dst_hbm.at[pl.ds(base, _EPW)], dst_v.at[pl.ds(0, _EPW)])
    pltpu.sync_copy(perm_hbm.at[:], perm_v)

    kvec = jnp.full((L,), K, jnp.int32)

    def nfill(i, _):
        nn_v[pl.ds(i * L, L)] = kvec
        return 0

    lax.fori_loop(0, (NPAD + 128) // L, nfill, 0)
    iota = lax.iota(jnp.int32, L)

    def pscatter(j, _):
        pv = perm_v[pl.ds(j * L, L)]
        plsc.store_scatter(nn_v, [pv], iota + j * L)
        return 0

    lax.fori_loop(0, 256 // L, pscatter, 0)

    z = jnp.zeros((L,), jnp.float32)

    def zbody(i, _):
        a2_v[i // (_A2 // L), pl.ds((i % (_A2 // L)) * L, L)] = z
        return 0

    lax.fori_loop(0, _A2 * (_A2 // L), zbody, 0)

    ones = jnp.full((L,), 1.0, jnp.float32)

    def body(i, _):
        for u in range(4):
            sv = src_v[pl.ds((i * 4 + u) * L, L)]
            dv = dst_v[pl.ds((i * 4 + u) * L, L)]
            sn = plsc.load_gather(nn_v, [sv])
            dn = plsc.load_gather(nn_v, [dv])
            m = (sn < K) & (dn < K)
            snc = jnp.minimum(sn, K)
            dnc = jnp.minimum(dn, K)
            plsc.addupdate_scatter(a2_v, [dnc, snc], ones, mask=m)
        return 0

    lax.fori_loop(0, _EPW_PAD // L // 4, body, 0)
    pltpu.sync_copy(a2_v, out_hbm.at[wid])


_RG = NPAD // 128


def _topk_sel_body(s_ref, poss_ref):
    s = s_ref[...]
    b = lax.bitcast_convert_type(s, jnp.uint32)
    neg = b >= jnp.uint32(0x80000000)
    key = jnp.where(neg, ~b, b | jnp.uint32(0x80000000))

    def bit_body(i, t):
        cand = t | (jnp.uint32(1) << (31 - i).astype(jnp.uint32))
        cnt = jnp.sum((key >= cand).astype(jnp.int32))
        return jnp.where(cnt >= K, cand, t)

    T = lax.fori_loop(0, 32, bit_body, jnp.uint32(0))
    gt = key > T
    eq = key == T
    cgt = jnp.sum(gt.astype(jnp.int32))

    i128 = lax.broadcasted_iota(jnp.int32, (128, 128), 0)
    j128 = lax.broadcasted_iota(jnp.int32, (128, 128), 1)
    minc = (i128 <= j128).astype(jnp.float32)
    irg = lax.broadcasted_iota(jnp.int32, (_RG, _RG), 0)
    jrg = lax.broadcasted_iota(jnp.int32, (_RG, _RG), 1)
    mex = (jrg < irg).astype(jnp.float32)

    def ex_prefix(m):
        cs = jnp.dot(m, minc, precision=lax.Precision.HIGHEST)
        rowtot = cs[:, 127:128]
        rowoff = jnp.dot(mex, rowtot, precision=lax.Precision.HIGHEST)
        return rowoff + cs - m

    eqf = eq.astype(jnp.float32)
    eq_ex = ex_prefix(eqf)
    fill = (K - cgt).astype(jnp.float32)
    keep = eq & (eq_ex < fill)
    sel = gt | keep
    self_f = sel.astype(jnp.float32)
    pos = ex_prefix(self_f)
    poss_ref[...] = jnp.where(sel, pos.astype(jnp.int32), -1)


def _topk_sel(score2d):
    return pl.pallas_call(
        _topk_sel_body,
        out_shape=jax.ShapeDtypeStruct((_RG, 128), jnp.int32),
    )(score2d)


_CB = 384
_NSL = NPAD // NS


@functools.partial(
    pl.kernel,
    out_type=(jax.ShapeDtypeStruct((_CB,), jnp.int32),
              jax.ShapeDtypeStruct((_CB,), jnp.float32)),
    mesh=_SC_MESH,
    scratch_types=[
        pltpu.VMEM((_NSL,), jnp.int32),
        pltpu.VMEM((_NSL,), jnp.float32),
        pltpu.VMEM((1, _CB), jnp.int32),
        pltpu.VMEM((1, _CB), jnp.float32),
        pltpu.VMEM((1, _CB), jnp.int32),
        pltpu.VMEM((1, _CB), jnp.float32),
        pltpu.VMEM((NS, _CB), jnp.int32),
        pltpu.VMEM((NS, _CB), jnp.float32),
        pltpu.VMEM_SHARED((NS, _CB), jnp.int32),
        pltpu.VMEM_SHARED((NS, _CB), jnp.float32),
        pltpu.VMEM_SHARED((NS, _CB), jnp.int32),
        pltpu.VMEM_SHARED((NS, _CB), jnp.float32),
    ],
    compiler_params=_SC_PARAMS,
)
def _sc_compact(poss_hbm, score_hbm, perm_hbm, vals_hbm, poss_v, score_v,
                compi_v, compv_v, permloc_v, valsloc_v, t16i_v, t16v_v, sh_i,
                sh_v, sh_p, sh_s):
    c = lax.axis_index("c")
    s = lax.axis_index("s")
    zero16 = jnp.zeros((L,), jnp.int32)
    zf = jnp.zeros((L,), jnp.float32)
    zi = jnp.zeros((L,), jnp.int32)

    def zboth(i, _):
        compi_v[0, pl.ds(i * L, L)] = zi
        compv_v[0, pl.ds(i * L, L)] = zf
        permloc_v[0, pl.ds(i * L, L)] = zi
        valsloc_v[0, pl.ds(i * L, L)] = zf
        return 0

    lax.fori_loop(0, _CB // L, zboth, 0)

    base = s * _NSL
    pltpu.sync_copy(poss_hbm.at[pl.ds(base, _NSL)], poss_v)
    pltpu.sync_copy(score_hbm.at[pl.ds(base, _NSL)], score_v)

    iota = lax.iota(jnp.int32, L)

    def scat(i, _):
        pv = poss_v[pl.ds(i * L, L)]
        m = pv >= 0
        pc = jnp.where(m, pv, _CB - 1)
        idxv = base + i * L + iota
        plsc.store_scatter(compi_v, [zero16, pc], idxv, mask=m)
        plsc.store_scatter(compv_v, [zero16, pc], score_v[pl.ds(i * L, L)],
                           mask=m)
        return 0

    lax.fori_loop(0, _NSL // L, scat, 0)
    pltpu.sync_copy(compi_v, sh_i.at[pl.ds(s, 1)])
    pltpu.sync_copy(compv_v, sh_v.at[pl.ds(s, 1)])
    plsc.subcore_barrier()
    pltpu.sync_copy(sh_i, t16i_v)
    pltpu.sync_copy(sh_v, t16v_v)

    def red(i, _):
        vi = zi
        vv = zf
        for t in range(NS):
            vi = vi + t16i_v[t, pl.ds(i * L, L)]
            vv = vv + t16v_v[t, pl.ds(i * L, L)]
        compi_v[0, pl.ds(i * L, L)] = vi
        compv_v[0, pl.ds(i * L, L)] = vv
        return 0

    lax.fori_loop(0, _CB // L, red, 0)

    nvr = (K + L - 1) // L
    lane0 = iota == 0

    for t in range(13):
        e = s * 13 + t
        ev = jnp.full((L,), 0, jnp.int32) + e
        ki = plsc.load_gather(compv_v, [zero16, ev])
        ii = plsc.load_gather(compi_v, [zero16, ev])
        cnt = jnp.zeros((L,), jnp.int32)
        for j in range(nvr):
            kv = compv_v[0, pl.ds(j * L, L)]
            iv = compi_v[0, pl.ds(j * L, L)]
            validj = (j * L + iota) < K
            m = ((kv > ki) | ((kv == ki) & (iv < ii))) & validj
            cnt = cnt + plsc.all_reduce_population_count(m)
        wm = lane0 & (e < K)
        plsc.store_scatter(permloc_v, [zero16, cnt], ii, mask=wm)
        plsc.store_scatter(valsloc_v, [zero16, cnt], ki, mask=wm)

    pltpu.sync_copy(permloc_v, sh_p.at[pl.ds(s, 1)])
    pltpu.sync_copy(valsloc_v, sh_s.at[pl.ds(s, 1)])
    plsc.subcore_barrier()

    @pl.when((s == 0) & (c == 0))
    def _():
        pltpu.sync_copy(sh_p, t16i_v)
        pltpu.sync_copy(sh_s, t16v_v)

        def red2(i, _):
            vi = zi
            vv = zf
            for t in range(NS):
                vi = vi + t16i_v[t, pl.ds(i * L, L)]
                vv = vv + t16v_v[t, pl.ds(i * L, L)]
            compi_v[0, pl.ds(i * L, L)] = vi
            compv_v[0, pl.ds(i * L, L)] = vv
            return 0

        lax.fori_loop(0, _CB // L, red2, 0)
        pltpu.sync_copy(compi_v.at[0], perm_hbm)
        pltpu.sync_copy(compv_v.at[0], vals_hbm)


def _final_body(cnt_ref, perm_ref, vals_ref, h_ref, w2_ref, b2_ref, o_ref):
    def merge(i, acc):
        return acc + cnt_ref[pl.ds(i * _A2, _A2), :]

    cnt = lax.fori_loop(1, NW, merge, cnt_ref[pl.ds(0, _A2), :])
    perm = perm_ref[...]
    vals = vals_ref[...]
    iota_n = lax.broadcasted_iota(jnp.int32, (_A2, NPAD), 1)
    oh = (iota_n == perm).astype(jnp.float32)
    hp = jnp.dot(oh, h_ref[...], precision=lax.Precision.HIGHEST)
    xp = hp * jnp.tanh(vals)
    h2 = jnp.dot(xp, w2_ref[...], preferred_element_type=jnp.float32)
    deg2 = jnp.sum(cnt, axis=1, keepdims=True) + 1.0
    dinv2 = lax.rsqrt(deg2)
    agg2 = dinv2 * jnp.dot(cnt, dinv2 * h2, precision=lax.Precision.HIGHEST)
    out = agg2 + h2 * (dinv2 * dinv2) + b2_ref[...]
    mx = jnp.max(out, axis=1, keepdims=True)
    ex = jnp.exp(out - mx)
    lse = jnp.log(jnp.sum(ex, axis=1, keepdims=True))
    o_ref[...] = out - mx - lse


def _final_tc(cnt_parts, perm208, vals208, hpad, W2, b2):
    return pl.pallas_call(
        _final_body,
        out_shape=jax.ShapeDtypeStruct((_A2, C), jnp.float32),
    )(cnt_parts, perm208, vals208, hpad, W2, b2)


def _mm_kernel(x_ref, w_ref, o_ref):
    o_ref[...] = jnp.dot(x_ref[...], w_ref[...],
                         preferred_element_type=jnp.float32)


def _matmul(x, w):
    return pl.pallas_call(
        _mm_kernel,
        out_shape=jax.ShapeDtypeStruct((x.shape[0], w.shape[1]), jnp.float32),
    )(x, w)


def kernel(x, edge_index, W1, b1, Wg_root, Wg_rel, bg, W2, b2):
    src = edge_index[0]
    dst = edge_index[1]

    deg_parts = _sc_deg(dst)
    deg = deg_parts.sum(axis=0)[:N] + 1.0
    dinv = 1.0 / jnp.sqrt(deg)

    h0 = _matmul(x, W1)
    g = h0 * dinv[:, None]
    gT = jnp.zeros((H, NPAD), jnp.float32).at[:, :N].set(g.T)

    msg_parts = _sc_msg(src, dst, gT)
    msgT = msg_parts.reshape(NW // 2, 2, 4, NPAD).sum(axis=0).reshape(H, NPAD)
    m = msgT[:, :N].T
    h = jax.nn.relu(dinv[:, None] * (m + g) + b1)

    hT = jnp.zeros((H, NPAD), jnp.float32).at[:, :N].set(h.T)
    agg_parts = _sc_msg(src, dst, hT)
    aggT = agg_parts.reshape(NW // 2, 2, 4, NPAD).sum(axis=0).reshape(H, NPAD)
    agg = aggT[:, :N].T
    score = (h @ Wg_root + agg @ Wg_rel + bg).squeeze(-1)

    score_pad = jnp.full((NPAD,), -3.4e38, jnp.float32).at[:N].set(score)
    poss = _topk_sel(score_pad.reshape(_RG, 128)).reshape(NPAD)
    perm_cb, vals_cb = _sc_compact(poss, score_pad)

    kmask = jnp.arange(256) < K
    perm256 = jnp.where(kmask, perm_cb[:256], NPAD)
    a2_parts = _sc_conv2(src, dst, perm256)

    perm208 = jnp.where(jnp.arange(_A2) < K, perm_cb[:_A2], NPAD).reshape(_A2, 1)
    vals208 = jnp.where(jnp.arange(_A2) < K, vals_cb[:_A2], 0.0).reshape(_A2, 1).astype(jnp.float32)
    hpad = jnp.zeros((NPAD, H), jnp.float32).at[:N].set(h)
    out = _final_tc(a2_parts.reshape(NW * _A2, _A2), perm208, vals208, hpad,
                    W2, b2.reshape(1, C))
    return out[:K]


# speedup vs baseline: 1.0176x; 1.0176x over previous
import functools

import jax
import jax.numpy as jnp
from jax import lax
from jax.experimental import pallas as pl
from jax.experimental.pallas import tpu as pltpu
from jax.experimental.pallas import tpu_sc as plsc

N = 10000
E = 160000
D = 256
H = 8
C = 32
K = 200

NPAD = 10240
NC, NS, L = 2, 16, 16
NW = NC * NS

_SC_MESH = plsc.VectorSubcoreMesh(core_axis_name="c", subcore_axis_name="s",
                                  num_cores=NC)
_SC_PARAMS = pltpu.CompilerParams(needs_layout_passes=False)


def _zero_vmem(ref, nwords):
    z = jnp.zeros((L,), ref.dtype)

    def body(i, _):
        ref[pl.ds(i * L, L)] = z
        return 0

    lax.fori_loop(0, nwords // L, body, 0)


_EPW = E // NW
_EPW_PAD = _EPW + 128 - (_EPW % 128)
_ACC_PAD = NPAD + 128


@functools.partial(
    pl.kernel,
    out_type=jax.ShapeDtypeStruct((NW, NPAD), jnp.float32),
    mesh=_SC_MESH,
    scratch_types=[
        pltpu.VMEM((_EPW_PAD,), jnp.int32),
        pltpu.VMEM((_ACC_PAD,), jnp.float32),
    ],
    compiler_params=_SC_PARAMS,
)
def _sc_deg(dst_hbm, out_hbm, idx_v, acc_v):
    wid = lax.axis_index("s") * NC + lax.axis_index("c")
    base = wid * _EPW
    dummy = jnp.full((L,), NPAD, jnp.int32)
    _tail0 = (_EPW // L) * L

    def fill(i, _):
        idx_v[pl.ds(_tail0 + i * L, L)] = dummy
        return 0

    lax.fori_loop(0, (_EPW_PAD - _tail0) // L, fill, 0)
    pltpu.sync_copy(dst_hbm.at[pl.ds(base, _EPW)], idx_v.at[pl.ds(0, _EPW)])
    _zero_vmem(acc_v, _ACC_PAD)
    ones = jnp.full((L,), 1.0, jnp.float32)

    def body(i, _):
        for u in range(4):
            idx = idx_v[pl.ds((i * 4 + u) * L, L)]
            plsc.addupdate_scatter(acc_v, [idx], ones)
        return 0

    lax.fori_loop(0, _EPW_PAD // L // 4, body, 0)
    pltpu.sync_copy(acc_v.at[pl.ds(0, NPAD)], out_hbm.at[wid])


_ECH = E // (NW // 2)


@functools.partial(
    pl.kernel,
    out_type=jax.ShapeDtypeStruct((NW, 4, NPAD), jnp.float32),
    mesh=_SC_MESH,
    scratch_types=[
        pltpu.VMEM((10240,), jnp.int32),
        pltpu.VMEM((10240,), jnp.int32),
        pltpu.VMEM((4, NPAD), jnp.float32),
        pltpu.VMEM((4, NPAD + 128), jnp.float32),
        pltpu.VMEM_SHARED((4, NPAD), jnp.float32),
    ],
    compiler_params=_SC_PARAMS,
)
def _sc_msg(src_hbm, dst_hbm, g_hbm, out_hbm, src_v, dst_v, g4_v, acc_v,
            g4sh_v):
    wid = lax.axis_index("s") * NC + lax.axis_index("c")
    g2 = wid % 2
    chunk = wid // 2
    base = chunk * _ECH
    zvec = jnp.zeros((L,), jnp.int32)
    dummy = jnp.full((L,), NPAD, jnp.int32)

    def fillt(i, _):
        src_v[pl.ds(_ECH + i * L, L)] = zvec
        dst_v[pl.ds(_ECH + i * L, L)] = dummy
        return 0

    lax.fori_loop(0, (10240 - _ECH) // L, fillt, 0)
    pltpu.sync_copy(src_hbm.at[pl.ds(base, _ECH)], src_v.at[pl.ds(0, _ECH)])
    pltpu.sync_copy(dst_hbm.at[pl.ds(base, _ECH)], dst_v.at[pl.ds(0, _ECH)])

    @pl.when(lax.axis_index("s") == 0)
    def _():
        pltpu.sync_copy(g_hbm.at[pl.ds(4 * g2, 4), :], g4sh_v)

    z = jnp.zeros((L,), jnp.float32)

    def zbody(i, _):
        acc_v[i // ((NPAD + 128) // L), pl.ds((i % ((NPAD + 128) // L)) * L, L)] = z
        return 0

    lax.fori_loop(0, 4 * ((NPAD + 128) // L), zbody, 0)
    plsc.subcore_barrier()
    pltpu.sync_copy(g4sh_v, g4_v)

    fvecs = [jnp.full((L,), f, jnp.int32) for f in range(4)]

    def body(i, _):
        for u in range(4):
            sv = src_v[pl.ds((i * 4 + u) * L, L)]
            dv = dst_v[pl.ds((i * 4 + u) * L, L)]
            for f in range(4):
                val = plsc.load_gather(g4_v, [fvecs[f], sv])
                plsc.addupdate_scatter(acc_v, [fvecs[f], dv], val)
        return 0

    lax.fori_loop(0, 10240 // L // 4, body, 0)
    pltpu.sync_copy(acc_v.at[:, pl.ds(0, NPAD)], out_hbm.at[wid])


_A2 = 208


@functools.partial(
    pl.kernel,
    out_type=jax.ShapeDtypeStruct((NW, _A2, _A2), jnp.float32),
    mesh=_SC_MESH,
    scratch_types=[
        pltpu.VMEM((_EPW_PAD,), jnp.int32),
        pltpu.VMEM((_EPW_PAD,), jnp.int32),
        pltpu.VMEM((NPAD + 128,), jnp.int32),
        pltpu.VMEM((256,), jnp.int32),
        pltpu.VMEM((_A2, _A2), jnp.float32),
    ],
    compiler_params=_SC_PARAMS,
)
def _sc_conv2(src_hbm, dst_hbm, perm_hbm, out_hbm, src_v, dst_v, nn_v, perm_v,
              a2_v):
    wid = lax.axis_index("s") * NC + lax.axis_index("c")
    base = wid * _EPW
    dummy = jnp.full((L,), NPAD, jnp.int32)
    _tail0 = (_EPW // L) * L

    def fill(i, _):
        src_v[pl.ds(_tail0 + i * L, L)] = dummy
        dst_v[pl.ds(_tail0 + i * L, L)] = dummy
        return 0

    lax.fori_loop(0, (_EPW_PAD - _tail0) // L, fill, 0)
    pltpu.sync_copy(src_hbm.at[pl.ds(base, _EPW)], src_v.at[pl.ds(0, _EPW)])
    pltpu.sync_copy(dst_hbm.at[pl.ds(base, _EPW)], dst_v.at[pl.ds(0, _EPW)])
    pltpu.sync_copy(perm_hbm.at[:], perm_v)

    kvec = jnp.full((L,), K, jnp.int32)

    def nfill(i, _):
        nn_v[pl.ds(i * L, L)] = kvec
        return 0

    lax.fori_loop(0, (NPAD + 128) // L, nfill, 0)
    iota = lax.iota(jnp.int32, L)

    def pscatter(j, _):
        pv = perm_v[pl.ds(j * L, L)]
        plsc.store_scatter(nn_v, [pv], iota + j * L)
        return 0

    lax.fori_loop(0, 256 // L, pscatter, 0)

    z = jnp.zeros((L,), jnp.float32)

    def zbody(i, _):
        a2_v[i // (_A2 // L), pl.ds((i % (_A2 // L)) * L, L)] = z
        return 0

    lax.fori_loop(0, _A2 * (_A2 // L), zbody, 0)

    ones = jnp.full((L,), 1.0, jnp.float32)

    def body(i, _):
        for u in range(4):
            sv = src_v[pl.ds((i * 4 + u) * L, L)]
            dv = dst_v[pl.ds((i * 4 + u) * L, L)]
            sn = plsc.load_gather(nn_v, [sv])
            dn = plsc.load_gather(nn_v, [dv])
            m = (sn < K) & (dn < K)
            snc = jnp.minimum(sn, K)
            dnc = jnp.minimum(dn, K)
            plsc.addupdate_scatter(a2_v, [dnc, snc], ones, mask=m)
        return 0

    lax.fori_loop(0, _EPW_PAD // L // 4, body, 0)
    pltpu.sync_copy(a2_v, out_hbm.at[wid])


_RG = NPAD // 128


def _topk_sel_body(s_ref, poss_ref):
    s = s_ref[...]
    b = lax.bitcast_convert_type(s, jnp.uint32)
    neg = b >= jnp.uint32(0x80000000)
    key = jnp.where(neg, ~b, b | jnp.uint32(0x80000000))

    def bit_body(i, t):
        cand = t | (jnp.uint32(1) << (31 - i).astype(jnp.uint32))
        cnt = jnp.sum((key >= cand).astype(jnp.int32))
        return jnp.where(cnt >= K, cand, t)

    T = lax.fori_loop(0, 32, bit_body, jnp.uint32(0))
    gt = key > T
    eq = key == T
    cgt = jnp.sum(gt.astype(jnp.int32))

    i128 = lax.broadcasted_iota(jnp.int32, (128, 128), 0)
    j128 = lax.broadcasted_iota(jnp.int32, (128, 128), 1)
    minc = (i128 <= j128).astype(jnp.float32)
    irg = lax.broadcasted_iota(jnp.int32, (_RG, _RG), 0)
    jrg = lax.broadcasted_iota(jnp.int32, (_RG, _RG), 1)
    mex = (jrg < irg).astype(jnp.float32)

    def ex_prefix(m):
        cs = jnp.dot(m, minc, precision=lax.Precision.HIGHEST)
        rowtot = cs[:, 127:128]
        rowoff = jnp.dot(mex, rowtot, precision=lax.Precision.HIGHEST)
        return rowoff + cs - m

    eqf = eq.astype(jnp.float32)
    eq_ex = ex_prefix(eqf)
    fill = (K - cgt).astype(jnp.float32)
    keep = eq & (eq_ex < fill)
    sel = gt | keep
    self_f = sel.astype(jnp.float32)
    pos = ex_prefix(self_f)
    poss_ref[...] = jnp.where(sel, pos.astype(jnp.int32), -1)


def _topk_sel(score2d):
    return pl.pallas_call(
        _topk_sel_body,
        out_shape=jax.ShapeDtypeStruct((_RG, 128), jnp.int32),
    )(score2d)


_CB = 384
_NSL = NPAD // NS


@functools.partial(
    pl.kernel,
    out_type=(jax.ShapeDtypeStruct((_CB,), jnp.int32),
              jax.ShapeDtypeStruct((_CB,), jnp.float32)),
    mesh=_SC_MESH,
    scratch_types=[
        pltpu.VMEM((_NSL,), jnp.int32),
        pltpu.VMEM((_NSL,), jnp.float32),
        pltpu.VMEM((1, _CB), jnp.int32),
        pltpu.VMEM((1, _CB), jnp.float32),
        pltpu.VMEM((1, _CB), jnp.int32),
        pltpu.VMEM((1, _CB), jnp.float32),
        pltpu.VMEM((NS, _CB), jnp.int32),
        pltpu.VMEM((NS, _CB), jnp.float32),
        pltpu.VMEM_SHARED((NS, _CB), jnp.int32),
        pltpu.VMEM_SHARED((NS, _CB), jnp.float32),
        pltpu.VMEM_SHARED((NS, _CB), jnp.int32),
        pltpu.VMEM_SHARED((NS, _CB), jnp.float32),
    ],
    compiler_params=_SC_PARAMS,
)
def _sc_compact(poss_hbm, score_hbm, perm_hbm, vals_hbm, poss_v, score_v,
                compi_v, compv_v, permloc_v, valsloc_v, t16i_v, t16v_v, sh_i,
                sh_v, sh_p, sh_s):
    c = lax.axis_index("c")
    s = lax.axis_index("s")
    zero16 = jnp.zeros((L,), jnp.int32)
    zf = jnp.zeros((L,), jnp.float32)
    zi = jnp.zeros((L,), jnp.int32)

    def zboth(i, _):
        compi_v[0, pl.ds(i * L, L)] = zi
        compv_v[0, pl.ds(i * L, L)] = zf
        permloc_v[0, pl.ds(i * L, L)] = zi
        valsloc_v[0, pl.ds(i * L, L)] = zf
        return 0

    lax.fori_loop(0, _CB // L, zboth, 0)

    base = s * _NSL
    pltpu.sync_copy(poss_hbm.at[pl.ds(base, _NSL)], poss_v)
    pltpu.sync_copy(score_hbm.at[pl.ds(base, _NSL)], score_v)

    iota = lax.iota(jnp.int32, L)

    def scat(i, _):
        pv = poss_v[pl.ds(i * L, L)]
        m = pv >= 0
        pc = jnp.where(m, pv, _CB - 1)
        idxv = base + i * L + iota
        plsc.store_scatter(compi_v, [zero16, pc], idxv, mask=m)
        plsc.store_scatter(compv_v, [zero16, pc], score_v[pl.ds(i * L, L)],
                           mask=m)
        return 0

    lax.fori_loop(0, _NSL // L, scat, 0)
    pltpu.sync_copy(compi_v, sh_i.at[pl.ds(s, 1)])
    pltpu.sync_copy(compv_v, sh_v.at[pl.ds(s, 1)])
    plsc.subcore_barrier()
    pltpu.sync_copy(sh_i, t16i_v)
    pltpu.sync_copy(sh_v, t16v_v)

    def red(i, _):
        vi = zi
        vv = zf
        for t in range(NS):
            vi = vi + t16i_v[t, pl.ds(i * L, L)]
            vv = vv + t16v_v[t, pl.ds(i * L, L)]
        compi_v[0, pl.ds(i * L, L)] = vi
        compv_v[0, pl.ds(i * L, L)] = vv
        return 0

    lax.fori_loop(0, _CB // L, red, 0)

    nvr = (K + L - 1) // L
    lane0 = iota == 0

    for t in range(13):
        e = s * 13 + t
        ev = jnp.full((L,), 0, jnp.int32) + e
        ki = plsc.load_gather(compv_v, [zero16, ev])
        ii = plsc.load_gather(compi_v, [zero16, ev])
        cnt = jnp.zeros((L,), jnp.int32)
        for j in range(nvr):
            kv = compv_v[0, pl.ds(j * L, L)]
            iv = compi_v[0, pl.ds(j * L, L)]
            validj = (j * L + iota) < K
            m = ((kv > ki) | ((kv == ki) & (iv < ii))) & validj
            cnt = cnt + plsc.all_reduce_population_count(m)
        wm = lane0 & (e < K)
        plsc.store_scatter(permloc_v, [zero16, cnt], ii, mask=wm)
        plsc.store_scatter(valsloc_v, [zero16, cnt], ki, mask=wm)

    pltpu.sync_copy(permloc_v, sh_p.at[pl.ds(s, 1)])
    pltpu.sync_copy(valsloc_v, sh_s.at[pl.ds(s, 1)])
    plsc.subcore_barrier()

    @pl.when((s == 0) & (c == 0))
    def _():
        pltpu.sync_copy(sh_p, t16i_v)
        pltpu.sync_copy(sh_s, t16v_v)

        def red2(i, _):
            vi = zi
            vv = zf
            for t in range(NS):
                vi = vi + t16i_v[t, pl.ds(i * L, L)]
                vv = vv + t16v_v[t, pl.ds(i * L, L)]
            compi_v[0, pl.ds(i * L, L)] = vi
            compv_v[0, pl.ds(i * L, L)] = vv
            return 0

        lax.fori_loop(0, _CB // L, red2, 0)
        pltpu.sync_copy(compi_v.at[0], perm_hbm)
        pltpu.sync_copy(compv_v.at[0], vals_hbm)


def _final_body(cnt_ref, perm_ref, vals_ref, h_ref, w2_ref, b2_ref, o_ref):
    def merge(i, acc):
        return acc + cnt_ref[pl.ds(i * _A2, _A2), :]

    cnt = lax.fori_loop(1, NW, merge, cnt_ref[pl.ds(0, _A2), :])
    perm = perm_ref[...]
    vals = vals_ref[...]
    iota_n = lax.broadcasted_iota(jnp.int32, (_A2, NPAD), 1)
    oh = (iota_n == perm).astype(jnp.float32)
    hp = jnp.dot(oh, h_ref[...], precision=lax.Precision.HIGHEST)
    xp = hp * jnp.tanh(vals)
    h2 = jnp.dot(xp, w2_ref[...], preferred_element_type=jnp.float32)
    deg2 = jnp.sum(cnt, axis=1, keepdims=True) + 1.0
    dinv2 = lax.rsqrt(deg2)
    agg2 = dinv2 * jnp.dot(cnt, dinv2 * h2, precision=lax.Precision.HIGHEST)
    out = agg2 + h2 * (dinv2 * dinv2) + b2_ref[...]
    mx = jnp.max(out, axis=1, keepdims=True)
    ex = jnp.exp(out - mx)
    lse = jnp.log(jnp.sum(ex, axis=1, keepdims=True))
    o_ref[...] = out - mx - lse


def _final_tc(cnt_parts, perm208, vals208, hpad, W2, b2):
    return pl.pallas_call(
        _final_body,
        out_shape=jax.ShapeDtypeStruct((_A2, C), jnp.float32),
    )(cnt_parts, perm208, vals208, hpad, W2, b2)


def _mm_kernel(x_ref, w_ref, o_ref):
    o_ref[...] = jnp.dot(x_ref[...], w_ref[...],
                         preferred_element_type=jnp.float32)


def _matmul(x, w):
    return pl.pallas_call(
        _mm_kernel,
        out_shape=jax.ShapeDtypeStruct((x.shape[0], w.shape[1]), jnp.float32),
    )(x, w)


def kernel(x, edge_index, W1, b1, Wg_root, Wg_rel, bg, W2, b2):
    src = edge_index[0]
    dst = edge_index[1]

    deg_parts = _sc_deg(dst)
    deg = deg_parts.sum(axis=0)[:N] + 1.0
    dinv = 1.0 / jnp.sqrt(deg)

    h0 = _matmul(x, W1)
    g = h0 * dinv[:, None]
    gT = jnp.zeros((H, NPAD), jnp.float32).at[:, :N].set(g.T)

    msg_parts = _sc_msg(src, dst, gT)
    msgT = msg_parts.reshape(NW // 2, 2, 4, NPAD).sum(axis=0).reshape(H, NPAD)
    m = msgT[:, :N].T
    h = jax.nn.relu(dinv[:, None] * (m + g) + b1)

    hT = jnp.zeros((H, NPAD), jnp.float32).at[:, :N].set(h.T)
    agg_parts = _sc_msg(src, dst, hT)
    aggT = agg_parts.reshape(NW // 2, 2, 4, NPAD).sum(axis=0).reshape(H, NPAD)
    agg = aggT[:, :N].T
    score = (h @ Wg_root + agg @ Wg_rel + bg).squeeze(-1)

    score_pad = jnp.full((NPAD,), -3.4e38, jnp.float32).at[:N].set(score)
    poss = _topk_sel(score_pad.reshape(_RG, 128)).reshape(NPAD)
    perm_cb, vals_cb = _sc_compact(poss, score_pad)

    kmask = jnp.arange(256) < K
    perm256 = jnp.where(kmask, perm_cb[:256], NPAD)
    a2_parts = _sc_conv2(src, dst, perm256)

    perm208 = jnp.where(jnp.arange(_A2) < K, perm_cb[:_A2], NPAD).reshape(_A2, 1)
    vals208 = jnp.where(jnp.arange(_A2) < K, vals_cb[:_A2], 0.0).reshape(_A2, 1).astype(jnp.float32)
    hpad = jnp.zeros((NPAD, H), jnp.float32).at[:N].set(h)
    out = _final_tc(a2_parts.reshape(NW * _A2, _A2), perm208, vals208, hpad,
                    W2, b2.reshape(1, C))
    return out[:K]
